# baseline (speedup 1.0000x reference)
import jax
import jax.numpy as jnp
from jax import lax
from jax.experimental import pallas as pl
from jax.experimental.pallas import tpu as pltpu

_N_CLASSES = 100000
_B = 2048
_SMOOTHING = 0.1
_BASE = _SMOOTHING / (_N_CLASSES - 1)
_CONF = 1.0 - _SMOOTHING

_BB = 32
_BC = 12800
_NJ = (_N_CLASSES + _BC - 1) // _BC
_NSTREAM = 2
_NI = _B // (_BB * _NSTREAM)
_ROWS_PER_STREAM = _B // _NSTREAM


def _gather_rows(t_sref, x_ref, row0, j, lane):
    gacc = jnp.zeros((1, 1), jnp.float32)
    for r in range(_BB):
        t = t_sref[row0 + r]
        off = ((t % _BC) // 128) * 128
        w = x_ref[pl.ds(r, 1), pl.ds(off, 128)]
        hit = (j * _BC + off + lane) == t
        gacc += jnp.sum(jnp.where(hit, w, 0.0)).reshape(1, 1)
    return gacc


def _body(t_sref, x0_ref, x1_ref, sum_ref, gsum_ref):
    i = pl.program_id(0)
    j = pl.program_id(1)

    @pl.when((i == 0) & (j == 0))
    def _init():
        sum_ref[...] = jnp.zeros((1, 1), jnp.float32)
        gsum_ref[...] = jnp.zeros((1, 1), jnp.float32)

    cols = j * _BC + lax.broadcasted_iota(jnp.int32, (1, _BC), 1)
    valid = cols < _N_CLASSES
    x0 = jnp.where(valid, x0_ref[...], 0.0)
    x1 = jnp.where(valid, x1_ref[...], 0.0)
    sum_ref[...] += (jnp.sum(x0) + jnp.sum(x1)).reshape(1, 1)

    lane = lax.broadcasted_iota(jnp.int32, (1, 128), 1)
    gsum_ref[...] += _gather_rows(t_sref, x0_ref, i * _BB, j, lane)
    gsum_ref[...] += _gather_rows(t_sref, x1_ref,
                                  _ROWS_PER_STREAM + i * _BB, j, lane)


def kernel(lsm, target):
    tgt = target.astype(jnp.int32)
    total, gsum = pl.pallas_call(
        _body,
        grid_spec=pltpu.PrefetchScalarGridSpec(
            num_scalar_prefetch=1,
            grid=(_NI, _NJ),
            in_specs=[
                pl.BlockSpec((_BB, _BC), lambda i, j, t: (i, j)),
                pl.BlockSpec((_BB, _BC), lambda i, j, t: (i + _NI, j)),
            ],
            out_specs=[
                pl.BlockSpec((1, 1), lambda i, j, t: (0, 0)),
                pl.BlockSpec((1, 1), lambda i, j, t: (0, 0)),
            ],
        ),
        out_shape=[
            jax.ShapeDtypeStruct((1, 1), jnp.float32),
            jax.ShapeDtypeStruct((1, 1), jnp.float32),
        ],
    )(tgt, lsm, lsm)
    scale = jnp.float32(_CONF - _BASE)
    return -(jnp.float32(_BASE) * total[0, 0] + scale * gsum[0, 0]) / jnp.float32(_B)


# speedup vs baseline: 3.4833x; 3.4833x over previous
import jax
import jax.numpy as jnp
from jax import lax
from jax.experimental import pallas as pl

_N_CLASSES = 100000
_B = 2048
_SMOOTHING = 0.1
_BASE = _SMOOTHING / (_N_CLASSES - 1)
_CONF = 1.0 - _SMOOTHING

_BR = 1000
_NBLK = _N_CLASSES // _BR


def _body(t_ref, x_ref, sum_ref, gsum_ref):
    j = pl.program_id(0)

    @pl.when(j == 0)
    def _init():
        sum_ref[...] = jnp.zeros((1, 1), jnp.float32)
        gsum_ref[...] = jnp.zeros((1, 1), jnp.float32)

    x = x_ref[...]
    rows = j * _BR + lax.broadcasted_iota(jnp.int32, (_BR, 1), 0)
    hit = rows == t_ref[...]
    sum_ref[...] += jnp.sum(x).reshape(1, 1)
    gsum_ref[...] += jnp.sum(jnp.where(hit, x, 0.0)).reshape(1, 1)


def kernel(lsm, target):
    t2d = target.astype(jnp.int32).reshape(1, _B)
    total, gsum = pl.pallas_call(
        _body,
        grid=(_NBLK,),
        in_specs=[
            pl.BlockSpec((1, _B), lambda j: (0, 0)),
            pl.BlockSpec((_BR, _B), lambda j: (j, 0)),
        ],
        out_specs=[
            pl.BlockSpec((1, 1), lambda j: (0, 0)),
            pl.BlockSpec((1, 1), lambda j: (0, 0)),
        ],
        out_shape=[
            jax.ShapeDtypeStruct((1, 1), jnp.float32),
            jax.ShapeDtypeStruct((1, 1), jnp.float32),
        ],
    )(t2d, lsm.T)
    scale = jnp.float32(_CONF - _BASE)
    return -(jnp.float32(_BASE) * total[0, 0] + scale * gsum[0, 0]) / jnp.float32(_B)


# speedup vs baseline: 3.8445x; 1.1037x over previous
import jax
import jax.numpy as jnp
from jax import lax
from jax.experimental import pallas as pl

_N_CLASSES = 100000
_B = 2048
_SMOOTHING = 0.1
_BASE = _SMOOTHING / (_N_CLASSES - 1)
_CONF = 1.0 - _SMOOTHING

_BR = 2000
_NBLK = _N_CLASSES // _BR


def _body(t_ref, x_ref, sum_ref, gsum_ref):
    j = pl.program_id(0)

    @pl.when(j == 0)
    def _init():
        sum_ref[...] = jnp.zeros((1, 1), jnp.float32)
        gsum_ref[...] = jnp.zeros((1, 1), jnp.float32)

    x = x_ref[...]
    rows = j * _BR + lax.broadcasted_iota(jnp.int32, (_BR, 1), 0)
    hit = rows == t_ref[...]
    sum_ref[...] += jnp.sum(x).reshape(1, 1)
    gsum_ref[...] += jnp.sum(jnp.where(hit, x, 0.0)).reshape(1, 1)


def kernel(lsm, target):
    t2d = target.astype(jnp.int32).reshape(1, _B)
    total, gsum = pl.pallas_call(
        _body,
        grid=(_NBLK,),
        in_specs=[
            pl.BlockSpec((1, _B), lambda j: (0, 0)),
            pl.BlockSpec((_BR, _B), lambda j: (j, 0)),
        ],
        out_specs=[
            pl.BlockSpec((1, 1), lambda j: (0, 0)),
            pl.BlockSpec((1, 1), lambda j: (0, 0)),
        ],
        out_shape=[
            jax.ShapeDtypeStruct((1, 1), jnp.float32),
            jax.ShapeDtypeStruct((1, 1), jnp.float32),
        ],
    )(t2d, lsm.T)
    scale = jnp.float32(_CONF - _BASE)
    return -(jnp.float32(_BASE) * total[0, 0] + scale * gsum[0, 0]) / jnp.float32(_B)


# speedup vs baseline: 3.9789x; 1.0350x over previous
import jax
import jax.numpy as jnp
from jax import lax
from jax.experimental import pallas as pl
from jax.experimental.pallas import tpu as pltpu

_N_CLASSES = 100000
_B = 2048
_SMOOTHING = 0.1
_BASE = _SMOOTHING / (_N_CLASSES - 1)
_CONF = 1.0 - _SMOOTHING

_BR = 4000
_NBLK = _N_CLASSES // _BR


def _body(t_ref, x_ref, sum_ref, gsum_ref):
    j = pl.program_id(0)

    @pl.when(j == 0)
    def _init():
        sum_ref[...] = jnp.zeros((1, 1), jnp.float32)
        gsum_ref[...] = jnp.zeros((1, 1), jnp.float32)

    x = x_ref[...]
    rows = j * _BR + lax.broadcasted_iota(jnp.int32, (_BR, 1), 0)
    hit = rows == t_ref[...]
    sum_ref[...] += jnp.sum(x).reshape(1, 1)
    gsum_ref[...] += jnp.sum(jnp.where(hit, x, 0.0)).reshape(1, 1)


def kernel(lsm, target):
    t2d = target.astype(jnp.int32).reshape(1, _B)
    total, gsum = pl.pallas_call(
        _body,
        grid=(_NBLK,),
        in_specs=[
            pl.BlockSpec((1, _B), lambda j: (0, 0)),
            pl.BlockSpec((_BR, _B), lambda j: (j, 0)),
        ],
        out_specs=[
            pl.BlockSpec((1, 1), lambda j: (0, 0)),
            pl.BlockSpec((1, 1), lambda j: (0, 0)),
        ],
        out_shape=[
            jax.ShapeDtypeStruct((1, 1), jnp.float32),
            jax.ShapeDtypeStruct((1, 1), jnp.float32),
        ],
        compiler_params=pltpu.CompilerParams(
            vmem_limit_bytes=128 * 1024 * 1024),
    )(t2d, lsm.T)
    scale = jnp.float32(_CONF - _BASE)
    return -(jnp.float32(_BASE) * total[0, 0] + scale * gsum[0, 0]) / jnp.float32(_B)
